# Initial kernel scaffold; baseline (speedup 1.0000x reference)
import jax
import jax.numpy as jnp
from jax.experimental import pallas as pl


def kernel(x, enc1_W, enc1_as, enc1_ad, enc1_b, enc2_W, enc2_as, enc2_ad, enc2_b, mu_W, mu_b, lv_W, lv_b, d1_W, d1_b, d2_W, d2_b, dec_W, dec_as, dec_ad, dec_b, edge_index):
    raise NotImplementedError("write your pallas kernel here")


# speedup vs baseline: 8.8798x; 8.8798x over previous
import functools

import jax
import jax.numpy as jnp
from jax import lax
from jax.experimental import pallas as pl
from jax.experimental.pallas import tpu as pltpu
from jax.experimental.pallas import tpu_sc as plsc

N = 10000
E = 160000
D_IN = 128
HID = 128
LAT = 32
D_OUT = 64
HEADS = 4

F = 128
ROWW = 144
NSUB = 16
C = 80
EPT = E // NSUB
NCH = EPT // C
GP = C // 16
NPAD = 10240
STRIP = NPAD // NSUB


@functools.lru_cache(maxsize=None)
def _make_sc_layer(nh):
    NADS = N * nh
    mesh = plsc.VectorSubcoreMesh(core_axis_name="c", subcore_axis_name="s",
                                  num_cores=2, num_subcores=NSUB)

    def body(tcat, adcat, src, dst, acc_out, hbuf, sbuf, dbuf, adbuf, accsh):
        c = lax.axis_index("c")
        s = lax.axis_index("s")
        zero16 = jnp.zeros((16,), jnp.float32)

        pltpu.sync_copy(adcat.at[pl.ds(c * NADS, NADS)], adbuf)

        def z_body(r, carry):
            for j in range(ROWW // 16):
                hbuf[r, pl.ds(j * 16, 16)] = zero16
            return carry
        lax.fori_loop(0, C, z_body, 0)
        sb = s * STRIP
        for j in range(STRIP // C):
            pltpu.sync_copy(hbuf, accsh.at[pl.ds(sb + j * C, C)])
        plsc.subcore_barrier()

        coff = c * N
        iota16 = lax.iota(jnp.int32, 16)

        def group_body(g, carry):
            rows = g * 16 + iota16
            dvec = dbuf[pl.ds(g * 16, 16)]
            ws = []
            for h in range(nh):
                colh = jnp.full((16,), F + h, jnp.int32)
                asrc = plsc.load_gather(hbuf, [rows, colh])
                adx = plsc.load_gather(adbuf, [dvec * nh + h])
                e = asrc + adx
                e = jnp.where(e >= 0.0, e, 0.2 * e)
                w = jnp.exp(e)
                plsc.store_scatter(hbuf, [rows, colh], w)
                ws.append(w)
            fh = F // nh
            for col in range(F):
                colv = jnp.full((16,), col, jnp.int32)
                v = plsc.load_gather(hbuf, [rows, colv])
                plsc.store_scatter(hbuf, [rows, colv], v * ws[col // fh])
            return carry

        def chunk_body(k, carry):
            base = s * EPT + k * C
            pltpu.sync_copy(src.at[pl.ds(base, C)], sbuf)
            pltpu.sync_copy(dst.at[pl.ds(base, C)], dbuf)

            def off_body(i, cr):
                sbuf[pl.ds(i * 16, 16)] = sbuf[pl.ds(i * 16, 16)] + coff
                return cr
            lax.fori_loop(0, C // 16, off_body, 0)

            pltpu.sync_copy(tcat.at[sbuf], hbuf)
            lax.fori_loop(0, GP, group_body, 0)
            pltpu.sync_copy(hbuf, accsh.at[dbuf], add=True)
            return carry

        lax.fori_loop(0, NCH, chunk_body, 0)
        plsc.subcore_barrier()
        pltpu.sync_copy(accsh.at[pl.ds(sb, STRIP)],
                        acc_out.at[c, pl.ds(sb, STRIP)])

    return pl.kernel(
        body,
        out_type=jax.ShapeDtypeStruct((2, NPAD, ROWW), jnp.float32),
        mesh=mesh,
        compiler_params=pltpu.CompilerParams(use_tc_tiling_on_sc=False,
                                             needs_layout_passes=False),
        scratch_types=[
            pltpu.VMEM((C, ROWW), jnp.float32),
            pltpu.VMEM((C,), jnp.int32),
            pltpu.VMEM((C,), jnp.int32),
            pltpu.VMEM((NADS,), jnp.float32),
            pltpu.VMEM_SHARED((NPAD, ROWW), jnp.float32),
        ],
    )


BN = 2000


def _dotf(a, b):
    return jnp.dot(a, b, preferred_element_type=jnp.float32)


def _k1_body(x_ref, w0, w1, w2, w3, wad, oa, ob, oad):
    x = x_ref[...]
    oa[0] = _dotf(x, w0[...])
    oa[1] = _dotf(x, w1[...])
    ob[0] = _dotf(x, w2[...])
    ob[1] = _dotf(x, w3[...])
    oad[...] = _dotf(x, wad[...])


def _enc1_pre(x, w0, w1, w2, w3, wad):
    return pl.pallas_call(
        _k1_body,
        grid=(N // BN,),
        in_specs=[
            pl.BlockSpec((BN, D_IN), lambda i: (i, 0)),
            pl.BlockSpec((D_IN, ROWW), lambda i: (0, 0)),
            pl.BlockSpec((D_IN, ROWW), lambda i: (0, 0)),
            pl.BlockSpec((D_IN, ROWW), lambda i: (0, 0)),
            pl.BlockSpec((D_IN, ROWW), lambda i: (0, 0)),
            pl.BlockSpec((D_IN, 16), lambda i: (0, 0)),
        ],
        out_specs=[
            pl.BlockSpec((2, BN, ROWW), lambda i: (0, i, 0)),
            pl.BlockSpec((2, BN, ROWW), lambda i: (0, i, 0)),
            pl.BlockSpec((BN, 16), lambda i: (i, 0)),
        ],
        out_shape=[
            jax.ShapeDtypeStruct((2, N, ROWW), jnp.float32),
            jax.ShapeDtypeStruct((2, N, ROWW), jnp.float32),
            jax.ShapeDtypeStruct((N, 16), jnp.float32),
        ],
    )(x, w0, w1, w2, w3, wad)


def _head_mean_1(acc_a, acc_b, bias):
    tot = None
    for a in (acc_a, acc_b):
        for cidx in (0, 1):
            hpart = a[cidx][:, 0:F] / (a[cidx][:, F:F + 1] + 1e-16)
            tot = hpart if tot is None else tot + hpart
    return tot * 0.25 + bias[...]


def _head_mean_2(acc, bias, oc):
    tot = None
    for cidx in (0, 1):
        a = acc[cidx]
        h0 = a[:, 0:oc] / (a[:, F:F + 1] + 1e-16)
        h1 = a[:, oc:2 * oc] / (a[:, F + 1:F + 2] + 1e-16)
        part = h0 + h1
        tot = part if tot is None else tot + part
    return tot * 0.25 + bias[...]


def _k2_body(acc_a, acc_b, b1, w0, w1, wad, oo, oad):
    hn = jax.nn.relu(_head_mean_1(acc_a, acc_b, b1))
    oo[0] = _dotf(hn, w0[...])
    oo[1] = _dotf(hn, w1[...])
    oad[...] = _dotf(hn, wad[...])


def _enc1_post_enc2_pre(acc_a, acc_b, b1, w0, w1, wad):
    return pl.pallas_call(
        _k2_body,
        grid=(N // BN,),
        in_specs=[
            pl.BlockSpec((2, BN, ROWW), lambda i: (0, i, 0)),
            pl.BlockSpec((2, BN, ROWW), lambda i: (0, i, 0)),
            pl.BlockSpec((1, HID), lambda i: (0, 0)),
            pl.BlockSpec((HID, ROWW), lambda i: (0, 0)),
            pl.BlockSpec((HID, ROWW), lambda i: (0, 0)),
            pl.BlockSpec((HID, 16), lambda i: (0, 0)),
        ],
        out_specs=[
            pl.BlockSpec((2, BN, ROWW), lambda i: (0, i, 0)),
            pl.BlockSpec((BN, 16), lambda i: (i, 0)),
        ],
        out_shape=[
            jax.ShapeDtypeStruct((2, N, ROWW), jnp.float32),
            jax.ShapeDtypeStruct((N, 16), jnp.float32),
        ],
    )(acc_a, acc_b, b1, w0, w1, wad)


def _k3_body(acc, b2, mu_w, mu_b, lv_w, lv_b, eps, d1_w, d1_b, d2_w, d2_b,
             w0, w1, wad, omu, olv, oo, oad):
    h2 = jax.nn.relu(_head_mean_2(acc, b2, HID // 2))
    mu = _dotf(h2, mu_w[...]) + mu_b[...]
    logvar = _dotf(h2, lv_w[...]) + lv_b[...]
    std = jnp.exp(0.5 * logvar)
    z = mu + eps[...] * std
    d = jax.nn.relu(_dotf(z, d1_w[...]) + d1_b[...])
    d = jax.nn.relu(_dotf(d, d2_w[...]) + d2_b[...])
    omu[...] = mu
    olv[...] = logvar
    oo[0] = _dotf(d, w0[...])
    oo[1] = _dotf(d, w1[...])
    oad[...] = _dotf(d, wad[...])


def _enc2_post_chain_dec_pre(acc, b2, mu_w, mu_b, lv_w, lv_b, eps,
                             d1_w, d1_b, d2_w, d2_b, w0, w1, wad):
    hh = HID // 2
    return pl.pallas_call(
        _k3_body,
        grid=(N // BN,),
        in_specs=[
            pl.BlockSpec((2, BN, ROWW), lambda i: (0, i, 0)),
            pl.BlockSpec((1, hh), lambda i: (0, 0)),
            pl.BlockSpec((hh, LAT), lambda i: (0, 0)),
            pl.BlockSpec((1, LAT), lambda i: (0, 0)),
            pl.BlockSpec((hh, LAT), lambda i: (0, 0)),
            pl.BlockSpec((1, LAT), lambda i: (0, 0)),
            pl.BlockSpec((BN, LAT), lambda i: (i, 0)),
            pl.BlockSpec((LAT, hh), lambda i: (0, 0)),
            pl.BlockSpec((1, hh), lambda i: (0, 0)),
            pl.BlockSpec((hh, HID), lambda i: (0, 0)),
            pl.BlockSpec((1, HID), lambda i: (0, 0)),
            pl.BlockSpec((HID, ROWW), lambda i: (0, 0)),
            pl.BlockSpec((HID, ROWW), lambda i: (0, 0)),
            pl.BlockSpec((HID, 16), lambda i: (0, 0)),
        ],
        out_specs=[
            pl.BlockSpec((BN, LAT), lambda i: (i, 0)),
            pl.BlockSpec((BN, LAT), lambda i: (i, 0)),
            pl.BlockSpec((2, BN, ROWW), lambda i: (0, i, 0)),
            pl.BlockSpec((BN, 16), lambda i: (i, 0)),
        ],
        out_shape=[
            jax.ShapeDtypeStruct((N, LAT), jnp.float32),
            jax.ShapeDtypeStruct((N, LAT), jnp.float32),
            jax.ShapeDtypeStruct((2, N, ROWW), jnp.float32),
            jax.ShapeDtypeStruct((N, 16), jnp.float32),
        ],
    )(acc, b2, mu_w, mu_b, lv_w, lv_b, eps, d1_w, d1_b, d2_w, d2_b,
      w0, w1, wad)


def _k4_body(acc, bd, orec):
    orec[...] = _head_mean_2(acc, bd, D_OUT)


def _dec_post(acc, bd):
    return pl.pallas_call(
        _k4_body,
        grid=(N // BN,),
        in_specs=[
            pl.BlockSpec((2, BN, ROWW), lambda i: (0, i, 0)),
            pl.BlockSpec((1, D_OUT), lambda i: (0, 0)),
        ],
        out_specs=pl.BlockSpec((BN, D_OUT), lambda i: (i, 0)),
        out_shape=jax.ShapeDtypeStruct((N, D_OUT), jnp.float32),
    )(acc, bd)


BI = 200


def _k5_body(a_ref, b_ref, o_ref):
    a = a_ref[...]
    b = b_ref[...]
    g = lax.dot_general(a, b, (((1,), (1,)), ((), ())),
                        preferred_element_type=jnp.float32)
    sqa = jnp.sum(a * a, axis=1, keepdims=True)
    sqb = jnp.sum(b * b, axis=1)[None, :]
    d2 = jnp.maximum(sqa + sqb - 2.0 * g, 0.0)
    o_ref[...] = jnp.sqrt(d2 + 1e-12)


def _pdist(recon):
    return pl.pallas_call(
        _k5_body,
        grid=(N // BI,),
        in_specs=[
            pl.BlockSpec((BI, D_OUT), lambda i: (i, 0)),
            pl.BlockSpec((N, D_OUT), lambda i: (0, 0)),
        ],
        out_specs=pl.BlockSpec((BI, N), lambda i: (i, 0)),
        out_shape=jax.ShapeDtypeStruct((N, N), jnp.float32),
    )(recon, recon)


def _pack_tables(W, att_src, att_dst, d_in, oc, nh):
    Wr = W.reshape(d_in, HEADS, oc)
    a_src_mat = jnp.einsum('dhc,hc->dh', Wr, att_src)
    a_dst_mat = jnp.einsum('dhc,hc->dh', Wr, att_dst)
    ngroups = HEADS // nh
    ws = []
    for gidx in range(ngroups):
        cols = [Wr[:, gidx * nh + h] for h in range(nh)]
        cols.append(a_src_mat[:, gidx * nh:(gidx + 1) * nh])
        cols.append(jnp.zeros((d_in, 16 - nh), jnp.float32))
        ws.append(jnp.concatenate(cols, axis=1))
    wad = jnp.pad(a_dst_mat, ((0, 0), (0, 16 - HEADS)))
    return ws, wad


def kernel(x, enc1_W, enc1_as, enc1_ad, enc1_b, enc2_W, enc2_as, enc2_ad,
           enc2_b, mu_W, mu_b, lv_W, lv_b, d1_W, d1_b, d2_W, d2_b, dec_W,
           dec_as, dec_ad, dec_b, edge_index):
    src = edge_index[0]
    dst = edge_index[1]

    (w0, w1, w2, w3), wad1 = _pack_tables(enc1_W, enc1_as, enc1_ad, D_IN,
                                          HID, 1)
    t_a, t_b, ad1 = _enc1_pre(x, w0, w1, w2, w3, wad1)
    acc_a = _make_sc_layer(1)(t_a.reshape(2 * N, ROWW),
                              jnp.concatenate([ad1[:, 0], ad1[:, 1]]), src, dst)
    t_b2, adcat_b, _ = lax.optimization_barrier(
        (t_b.reshape(2 * N, ROWW),
         jnp.concatenate([ad1[:, 2], ad1[:, 3]]), acc_a))
    acc_b = _make_sc_layer(1)(t_b2, adcat_b, src, dst)

    (w20, w21), wad2 = _pack_tables(enc2_W, enc2_as, enc2_ad, HID,
                                    HID // 2, 2)
    t2, ad2 = _enc1_post_enc2_pre(acc_a, acc_b, enc1_b.reshape(1, -1),
                                  w20, w21, wad2)
    adcat2 = jnp.concatenate([ad2[:, 0:2].reshape(-1),
                              ad2[:, 2:4].reshape(-1)])
    acc2 = _make_sc_layer(2)(t2.reshape(2 * N, ROWW), adcat2, src, dst)

    eps = jax.random.normal(jax.random.key(42), (N, LAT), dtype=jnp.float32)
    (wd0, wd1), wadd = _pack_tables(dec_W, dec_as, dec_ad, HID, D_OUT, 2)
    mu, logvar, td, add = _enc2_post_chain_dec_pre(
        acc2, enc2_b.reshape(1, -1), mu_W, mu_b.reshape(1, -1), lv_W,
        lv_b.reshape(1, -1), eps, d1_W, d1_b.reshape(1, -1), d2_W,
        d2_b.reshape(1, -1), wd0, wd1, wadd)
    adcatd = jnp.concatenate([add[:, 0:2].reshape(-1),
                              add[:, 2:4].reshape(-1)])
    accd = _make_sc_layer(2)(td.reshape(2 * N, ROWW), adcatd, src, dst)

    recon = _dec_post(accd, dec_b.reshape(1, -1))
    pdist = _pdist(recon)
    return (recon, mu, logvar, pdist)
